# baseline (speedup 1.0000x reference)
import functools

import jax
import jax.numpy as jnp
from jax import lax
from jax.experimental import pallas as pl
from jax.experimental.pallas import tpu as pltpu
from jax.experimental.pallas import tpu_sc as plsc

_NC = 2
_NS = 16
_L = 16
_NW = _NC * _NS


@functools.lru_cache(maxsize=None)
def _build_sc_add(B, S, D, CH):
    seq_per_w = S // _NW
    n_chunks = seq_per_w // CH
    n_elem = CH * D
    mesh = plsc.VectorSubcoreMesh(
        core_axis_name="c", subcore_axis_name="s",
        num_cores=_NC, num_subcores=_NS)

    @functools.partial(
        pl.kernel,
        out_type=jax.ShapeDtypeStruct((B * S * D,), jnp.float32),
        mesh=mesh,
        scratch_types=[
            pltpu.VMEM((n_elem,), jnp.float32),
            pltpu.VMEM((n_elem,), jnp.float32),
        ],
    )
    def run(x_hbm, pe_hbm, out_hbm, buf_pe, buf_x):
        wid = lax.axis_index("s") * _NC + lax.axis_index("c")
        s_base = wid * seq_per_w
        for t in range(n_chunks):
            p0 = (s_base + t * CH) * D
            pltpu.sync_copy(pe_hbm.at[pl.ds(p0, n_elem)], buf_pe)
            for b in range(B):
                r0 = b * S * D + p0
                pltpu.sync_copy(x_hbm.at[pl.ds(r0, n_elem)], buf_x)

                @plsc.parallel_loop(0, n_elem, step=_L)
                def _(i):
                    plsc.addupdate(buf_x.at[pl.ds(i, _L)],
                                   buf_pe[pl.ds(i, _L)])

                pltpu.sync_copy(buf_x, out_hbm.at[pl.ds(r0, n_elem)])

    return run


def kernel(x, pe):
    B, S, D = x.shape
    run = _build_sc_add(B, S, D, 32)
    out = run(x.reshape(-1), pe.reshape(-1))
    return out.reshape(B, S, D)


# speedup vs baseline: 4.0106x; 4.0106x over previous
import functools

import jax
import jax.numpy as jnp
from jax import lax
from jax.experimental import pallas as pl
from jax.experimental.pallas import tpu as pltpu
from jax.experimental.pallas import tpu_sc as plsc

_NC = 2
_NS = 16
_L = 16
_NW = _NC * _NS


@functools.lru_cache(maxsize=None)
def _build_sc_add(B, S, D, CH):
    seq_per_w = S // _NW
    n_chunks = seq_per_w // CH
    n_col = D // _L
    mesh = plsc.VectorSubcoreMesh(
        core_axis_name="c", subcore_axis_name="s",
        num_cores=_NC, num_subcores=_NS)

    @functools.partial(
        pl.kernel,
        out_type=jax.ShapeDtypeStruct((B, S, D), jnp.float32),
        mesh=mesh,
        scratch_types=[
            pltpu.VMEM((CH, D), jnp.float32),
            pltpu.VMEM((CH, D), jnp.float32),
        ],
    )
    def run(x_hbm, pe_hbm, out_hbm, buf_pe, buf_x):
        wid = lax.axis_index("s") * _NC + lax.axis_index("c")
        s_base = wid * seq_per_w

        def chunk_body(t, carry):
            p0 = s_base + t * CH
            pltpu.sync_copy(pe_hbm.at[pl.ds(p0, CH)], buf_pe)

            def batch_body(b, carry):
                pltpu.sync_copy(x_hbm.at[b, pl.ds(p0, CH)], buf_x)

                G = 8

                @plsc.parallel_loop(0, CH)
                def _(r):
                    for g in range(n_col // G):
                        cols = [(g * G + j) * _L for j in range(G)]
                        vs = [buf_pe[r, pl.ds(c, _L)] for c in cols]
                        for c, v in zip(cols, vs):
                            plsc.addupdate(buf_x.at[r, pl.ds(c, _L)], v)

                pltpu.sync_copy(buf_x, out_hbm.at[b, pl.ds(p0, CH)])
                return carry

            return lax.fori_loop(0, B, batch_body, carry)

        lax.fori_loop(0, n_chunks, chunk_body, 0)

    return run


def kernel(x, pe):
    B, S, D = x.shape
    run = _build_sc_add(B, S, D, 32)
    return run(x, pe)


# speedup vs baseline: 4.0283x; 1.0044x over previous
import functools

import jax
import jax.numpy as jnp
from jax import lax
from jax.experimental import pallas as pl
from jax.experimental.pallas import tpu as pltpu
from jax.experimental.pallas import tpu_sc as plsc

_NC = 2
_NS = 16
_L = 16
_NW = _NC * _NS


@functools.lru_cache(maxsize=None)
def _build_sc_add(B, S, D, CH):
    seq_per_w = S // _NW
    n_chunks = seq_per_w // CH
    n_col = D // _L
    G = 8
    mesh = plsc.VectorSubcoreMesh(
        core_axis_name="c", subcore_axis_name="s",
        num_cores=_NC, num_subcores=_NS)

    @functools.partial(
        pl.kernel,
        out_type=jax.ShapeDtypeStruct((B, S, D), jnp.float32),
        mesh=mesh,
        scratch_types=[
            pltpu.VMEM((2, B, CH, D), jnp.float32),
            pltpu.VMEM((2, CH, D), jnp.float32),
            pltpu.SemaphoreType.DMA((2, B)),
            pltpu.SemaphoreType.DMA((2, B)),
            pltpu.SemaphoreType.DMA((2,)),
        ],
    )
    def run(x_hbm, pe_hbm, out_hbm, x_sl, pe_sl, in_sems, out_sems,
            pe_sems):
        wid = lax.axis_index("s") * _NC + lax.axis_index("c")
        s_base = wid * seq_per_w

        def start_pe(t, p):
            pltpu.async_copy(pe_hbm.at[pl.ds(s_base + t * CH, CH)],
                             pe_sl.at[p], pe_sems.at[p])

        def wait_pe(t, p):
            pltpu.make_async_copy(pe_hbm.at[pl.ds(s_base + t * CH, CH)],
                                  pe_sl.at[p], pe_sems.at[p]).wait()

        def start_in(t, p, b):
            pltpu.async_copy(x_hbm.at[b, pl.ds(s_base + t * CH, CH)],
                             x_sl.at[p, b], in_sems.at[p, b])

        def wait_in(t, p, b):
            pltpu.make_async_copy(x_hbm.at[b, pl.ds(s_base + t * CH, CH)],
                                  x_sl.at[p, b], in_sems.at[p, b]).wait()

        def start_out(t, p, b):
            pltpu.async_copy(x_sl.at[p, b],
                             out_hbm.at[b, pl.ds(s_base + t * CH, CH)],
                             out_sems.at[p, b])

        def wait_out(t, p, b):
            pltpu.make_async_copy(x_sl.at[p, b],
                                  out_hbm.at[b, pl.ds(s_base + t * CH, CH)],
                                  out_sems.at[p, b]).wait()

        start_pe(0, 0)
        for b in range(B):
            start_in(0, 0, b)
        start_pe(1, 1)

        def body(t, carry):
            p = lax.rem(t, 2)
            q = 1 - p
            wait_pe(t, p)
            for b in range(B):
                wait_in(t, p, b)

            @plsc.parallel_loop(0, CH)
            def _(r):
                for g in range(n_col // G):
                    cols = [(g * G + j) * _L for j in range(G)]
                    pe_vs = [pe_sl[p, r, pl.ds(c, _L)] for c in cols]
                    for b in range(B):
                        xs = [x_sl[p, b, r, pl.ds(c, _L)] for c in cols]
                        for c, xv, pv in zip(cols, xs, pe_vs):
                            x_sl[p, b, r, pl.ds(c, _L)] = xv + pv

            @pl.when(t < n_chunks - 1)
            def _():
                for b in range(B):
                    @pl.when(t > 0)
                    def _():
                        wait_out(t - 1, q, b)
                    start_in(t + 1, q, b)

            for b in range(B):
                start_out(t, p, b)

            @pl.when(t < n_chunks - 2)
            def _():
                start_pe(t + 2, p)
            return carry

        lax.fori_loop(0, n_chunks, body, 0)

        p_last = (n_chunks - 1) % 2
        for b in range(B):
            wait_out(n_chunks - 1, p_last, b)

    return run


def kernel(x, pe):
    B, S, D = x.shape
    run = _build_sc_add(B, S, D, 8)
    return run(x, pe)


# speedup vs baseline: 4.8972x; 1.2157x over previous
import functools

import jax
import jax.numpy as jnp
from jax import lax
from jax.experimental import pallas as pl
from jax.experimental.pallas import tpu as pltpu
from jax.experimental.pallas import tpu_sc as plsc

_NC = 2
_NS = 16
_L = 16
_NW = _NC * _NS


@functools.lru_cache(maxsize=None)
def _build_sc_add(B, S, D, CH):
    seq_per_w = S // _NW
    n_chunks = seq_per_w // CH
    n_col = D // _L
    G = 8
    mesh = plsc.VectorSubcoreMesh(
        core_axis_name="c", subcore_axis_name="s",
        num_cores=_NC, num_subcores=_NS)

    @functools.partial(
        pl.kernel,
        out_type=jax.ShapeDtypeStruct((B, S, D), jnp.float32),
        mesh=mesh,
        scratch_types=[
            pltpu.VMEM((3, B, CH, D), jnp.float32),
            pltpu.VMEM((2, CH, D), jnp.float32),
            pltpu.SemaphoreType.DMA((3, B)),
            pltpu.SemaphoreType.DMA((3, B)),
            pltpu.SemaphoreType.DMA((2,)),
        ],
    )
    def run(x_hbm, pe_hbm, out_hbm, x_sl, pe_sl, in_sems, out_sems,
            pe_sems):
        wid = lax.axis_index("s") * _NC + lax.axis_index("c")
        s_base = wid * seq_per_w

        def start_pe(t, p):
            pltpu.async_copy(pe_hbm.at[pl.ds(s_base + t * CH, CH)],
                             pe_sl.at[p], pe_sems.at[p])

        def wait_pe(t, p):
            pltpu.make_async_copy(pe_hbm.at[pl.ds(s_base + t * CH, CH)],
                                  pe_sl.at[p], pe_sems.at[p]).wait()

        def start_in(t, p, b):
            pltpu.async_copy(x_hbm.at[b, pl.ds(s_base + t * CH, CH)],
                             x_sl.at[p, b], in_sems.at[p, b])

        def wait_in(t, p, b):
            pltpu.make_async_copy(x_hbm.at[b, pl.ds(s_base + t * CH, CH)],
                                  x_sl.at[p, b], in_sems.at[p, b]).wait()

        def start_out(t, p, b):
            pltpu.async_copy(x_sl.at[p, b],
                             out_hbm.at[b, pl.ds(s_base + t * CH, CH)],
                             out_sems.at[p, b])

        def wait_out(t, p, b):
            pltpu.make_async_copy(x_sl.at[p, b],
                                  out_hbm.at[b, pl.ds(s_base + t * CH, CH)],
                                  out_sems.at[p, b]).wait()

        start_pe(0, 0)
        for b in range(B):
            start_in(0, 0, b)
        start_pe(1, 1)
        for b in range(B):
            start_in(1, 1, b)

        def body(t, carry):
            p = lax.rem(t, 3)
            pp = lax.rem(t, 2)
            q = lax.rem(t + 2, 3)
            wait_pe(t, pp)
            for b in range(B):
                wait_in(t, p, b)

            @plsc.parallel_loop(0, CH)
            def _(r):
                for g in range(n_col // G):
                    cols = [(g * G + j) * _L for j in range(G)]
                    pe_vs = [pe_sl[pp, r, pl.ds(c, _L)] for c in cols]
                    for b in range(B):
                        xs = [x_sl[p, b, r, pl.ds(c, _L)] for c in cols]
                        for c, xv, pv in zip(cols, xs, pe_vs):
                            x_sl[p, b, r, pl.ds(c, _L)] = xv + pv

            @pl.when(t < n_chunks - 2)
            def _():
                for b in range(B):
                    @pl.when(t > 0)
                    def _():
                        wait_out(t - 1, q, b)
                    start_in(t + 2, q, b)

            for b in range(B):
                start_out(t, p, b)

            @pl.when(t < n_chunks - 2)
            def _():
                start_pe(t + 2, pp)
            return carry

        lax.fori_loop(0, n_chunks, body, 0)

        for t in range(n_chunks - 3, n_chunks):
            for b in range(B):
                wait_out(t, t % 3, b)

    return run


def kernel(x, pe):
    B, S, D = x.shape
    run = _build_sc_add(B, S, D, 8)
    return run(x, pe)
